# Initial kernel scaffold; baseline (speedup 1.0000x reference)
import jax
import jax.numpy as jnp
from jax.experimental import pallas as pl


def kernel(dense_fea, W, b):
    raise NotImplementedError("write your pallas kernel here")


# speedup vs baseline: 7.9766x; 7.9766x over previous
import functools

import jax
import jax.numpy as jnp
from jax import lax
from jax.experimental import pallas as pl
from jax.experimental.pallas import tpu as pltpu
from jax.experimental.pallas import tpu_sc as plsc

_K = 10
_KPAD = 16


def _prep_body(x_ref, w_ref, b_ref, idx_ref, g_ref, c_ref):
    x = x_ref[0]
    d = x.shape[0]
    n = x.shape[1]
    w1 = w_ref[:, :d]
    w2 = w_ref[:, d:]

    xx = lax.dot_general(x, x, (((0,), (0,)), ((), ())),
                         preferred_element_type=jnp.float32)
    sq_row = jnp.sum(x * x, axis=0, keepdims=True)
    xt = jnp.transpose(x, (1, 0))
    sq_col = jnp.sum(xt * xt, axis=1, keepdims=True)
    m = -(sq_col - 2.0 * xx + sq_row)

    lane = lax.broadcasted_iota(jnp.int32, (n, n), 1)
    lane16 = lax.broadcasted_iota(jnp.int32, (n, _KPAD), 1)
    base = pl.program_id(0) * n
    neg = jnp.float32(-jnp.inf)
    big = jnp.int32(2**30)
    idxacc = jnp.zeros((n, _KPAD), jnp.int32)
    for t in range(_K):
        vmax = jnp.max(m, axis=1, keepdims=True)
        cand = jnp.where(m >= vmax, lane, big)
        amin = jnp.min(cand, axis=1, keepdims=True)
        idxacc = jnp.where(lane16 == t, amin + base, idxacc)
        m = jnp.where(lane == amin, neg, m)
    idx_ref[0] = idxacc

    g_ref[0] = lax.dot_general(x, w2, (((0,), (1,)), ((), ())),
                               preferred_element_type=jnp.float32,
                               precision=lax.Precision.HIGHEST)
    c_ref[0] = lax.dot_general(w1 - w2, x, (((1,), (0,)), ((), ())),
                               preferred_element_type=jnp.float32,
                               precision=lax.Precision.HIGHEST) + b_ref[...]


def _epi_body(m_ref, c_ref, o_ref):
    z = c_ref[0] + jnp.transpose(m_ref[0], (1, 0))
    o_ref[0] = jnp.maximum(z, 0.2 * z)


def _prep_call(x, w, b2):
    bs, d, n = x.shape
    o = w.shape[0]
    return pl.pallas_call(
        _prep_body,
        grid=(bs,),
        in_specs=[
            pl.BlockSpec((1, d, n), lambda i: (i, 0, 0)),
            pl.BlockSpec((o, 2 * d), lambda i: (0, 0)),
            pl.BlockSpec((o, 1), lambda i: (0, 0)),
        ],
        out_specs=[
            pl.BlockSpec((1, n, _KPAD), lambda i: (i, 0, 0)),
            pl.BlockSpec((1, n, o), lambda i: (i, 0, 0)),
            pl.BlockSpec((1, o, n), lambda i: (i, 0, 0)),
        ],
        out_shape=[
            jax.ShapeDtypeStruct((bs, n, _KPAD), jnp.int32),
            jax.ShapeDtypeStruct((bs, n, o), jnp.float32),
            jax.ShapeDtypeStruct((bs, o, n), jnp.float32),
        ],
    )(x, w, b2)


def _epi_call(m, c):
    bs, n, o = m.shape
    return pl.pallas_call(
        _epi_body,
        grid=(bs,),
        in_specs=[
            pl.BlockSpec((1, n, o), lambda i: (i, 0, 0)),
            pl.BlockSpec((1, o, n), lambda i: (i, 0, 0)),
        ],
        out_specs=pl.BlockSpec((1, o, n), lambda i: (i, 0, 0)),
        out_shape=jax.ShapeDtypeStruct((bs, o, n), jnp.float32),
    )(m, c)


@functools.lru_cache(maxsize=None)
def _gather_max_call(rows, o):
    ncores, nsub = 2, 16
    nw = ncores * nsub
    rows_per_w = rows // nw
    r_chunk = 8
    n_chunks = rows_per_w // r_chunk
    nlane = 16
    mesh = plsc.VectorSubcoreMesh(core_axis_name="c", subcore_axis_name="s")

    @functools.partial(
        pl.kernel,
        mesh=mesh,
        out_type=jax.ShapeDtypeStruct((rows, o), jnp.float32),
        scratch_types=[
            pltpu.VMEM((r_chunk * _K,), jnp.int32),
            pltpu.VMEM((r_chunk * _K, o), jnp.float32),
            pltpu.VMEM((r_chunk, o), jnp.float32),
            pltpu.SemaphoreType.DMA,
        ],
    )
    def gather_max(g_hbm, idx_hbm, m_hbm, idx_v, rows_v, out_v, sem):
        wid = lax.axis_index("s") * ncores + lax.axis_index("c")

        @pl.loop(0, n_chunks)
        def _(ch):
            rowbase = wid * rows_per_w + ch * r_chunk
            pltpu.sync_copy(idx_hbm.at[pl.ds(rowbase * _K, r_chunk * _K)],
                            idx_v)
            pltpu.async_copy(g_hbm.at[idx_v], rows_v, sem).wait()
            for r in range(r_chunk):
                for cc in range(o // nlane):
                    sl = pl.ds(cc * nlane, nlane)
                    acc = rows_v[r * _K, sl]
                    for t in range(1, _K):
                        acc = jnp.maximum(acc, rows_v[r * _K + t, sl])
                    out_v[r, sl] = acc
            pltpu.sync_copy(out_v, m_hbm.at[pl.ds(rowbase, r_chunk)])

    return gather_max


def kernel(dense_fea, W, b):
    bs, emb, n_stk, n_stk_pnt = dense_fea.shape
    n = n_stk * n_stk_pnt
    o = W.shape[0]
    x = dense_fea.reshape(bs, emb, n)
    idx, g, c = _prep_call(x, W, b.reshape(o, 1))
    idx_flat = idx[:, :, :_K].reshape(bs * n * _K)
    g_flat = g.reshape(bs * n, o)
    m = _gather_max_call(bs * n, o)(g_flat, idx_flat)
    out = _epi_call(m.reshape(bs, n, o), c)
    return out.reshape(bs, o, n_stk, n_stk_pnt)


# speedup vs baseline: 9.1587x; 1.1482x over previous
import functools

import jax
import jax.numpy as jnp
from jax import lax
from jax.experimental import pallas as pl
from jax.experimental.pallas import tpu as pltpu
from jax.experimental.pallas import tpu_sc as plsc

_K = 10
_KPAD = 16


def _prep_body(x_ref, w_ref, b_ref, idx_ref, g_ref, c_ref):
    x = x_ref[0]
    d = x.shape[0]
    n = x.shape[1]
    w1 = w_ref[:, :d]
    w2 = w_ref[:, d:]

    xx = lax.dot_general(x, x, (((0,), (0,)), ((), ())),
                         preferred_element_type=jnp.float32)
    sq_row = jnp.sum(x * x, axis=0, keepdims=True)
    xt = jnp.transpose(x, (1, 0))
    sq_col = jnp.sum(xt * xt, axis=1, keepdims=True)
    m = -(sq_col - 2.0 * xx + sq_row)

    lane = lax.broadcasted_iota(jnp.int32, (n, n), 1)
    lane16 = lax.broadcasted_iota(jnp.int32, (n, _KPAD), 1)
    base = pl.program_id(0) * n
    neg = jnp.float32(-jnp.inf)
    big = jnp.int32(2**30)
    idxacc = jnp.zeros((n, _KPAD), jnp.int32)
    for t in range(_K):
        vmax = jnp.max(m, axis=1, keepdims=True)
        cand = jnp.where(m >= vmax, lane, big)
        amin = jnp.min(cand, axis=1, keepdims=True)
        idxacc = jnp.where(lane16 == t, amin + base, idxacc)
        m = jnp.where(lane == amin, neg, m)
    idx_ref[0] = idxacc

    g_ref[0] = lax.dot_general(x, w2, (((0,), (1,)), ((), ())),
                               preferred_element_type=jnp.float32,
                               precision=lax.Precision.HIGHEST)
    c_ref[0] = lax.dot_general(w1 - w2, x, (((1,), (0,)), ((), ())),
                               preferred_element_type=jnp.float32,
                               precision=lax.Precision.HIGHEST) + b_ref[...]


def _epi_body(m_ref, c_ref, o_ref):
    z = c_ref[0] + jnp.transpose(m_ref[0], (1, 0))
    o_ref[0] = jnp.maximum(z, 0.2 * z)


def _prep_call(x, w, b2):
    bs, d, n = x.shape
    o = w.shape[0]
    return pl.pallas_call(
        _prep_body,
        grid=(bs,),
        in_specs=[
            pl.BlockSpec((1, d, n), lambda i: (i, 0, 0)),
            pl.BlockSpec((o, 2 * d), lambda i: (0, 0)),
            pl.BlockSpec((o, 1), lambda i: (0, 0)),
        ],
        out_specs=[
            pl.BlockSpec((1, n, _KPAD), lambda i: (i, 0, 0)),
            pl.BlockSpec((1, n, o), lambda i: (i, 0, 0)),
            pl.BlockSpec((1, o, n), lambda i: (i, 0, 0)),
        ],
        out_shape=[
            jax.ShapeDtypeStruct((bs, n, _KPAD), jnp.int32),
            jax.ShapeDtypeStruct((bs, n, o), jnp.float32),
            jax.ShapeDtypeStruct((bs, o, n), jnp.float32),
        ],
    )(x, w, b2)


def _epi_call(m, c):
    bs, n, o = m.shape
    return pl.pallas_call(
        _epi_body,
        grid=(bs,),
        in_specs=[
            pl.BlockSpec((1, n, o), lambda i: (i, 0, 0)),
            pl.BlockSpec((1, o, n), lambda i: (i, 0, 0)),
        ],
        out_specs=pl.BlockSpec((1, o, n), lambda i: (i, 0, 0)),
        out_shape=jax.ShapeDtypeStruct((bs, o, n), jnp.float32),
    )(m, c)


@functools.lru_cache(maxsize=None)
def _gather_max_call(rows, o):
    ncores, nsub = 2, 16
    nw = ncores * nsub
    rows_per_w = rows // nw
    r_chunk = 8
    n_chunks = rows_per_w // r_chunk
    nlane = 16
    mesh = plsc.VectorSubcoreMesh(core_axis_name="c", subcore_axis_name="s")

    @functools.partial(
        pl.kernel,
        mesh=mesh,
        out_type=jax.ShapeDtypeStruct((rows, o), jnp.float32),
        scratch_types=[
            pltpu.VMEM((rows_per_w * _K,), jnp.int32),
            pltpu.VMEM((2, r_chunk * _K, o), jnp.float32),
            pltpu.VMEM((2, r_chunk, o), jnp.float32),
            pltpu.SemaphoreType.DMA,
            pltpu.SemaphoreType.DMA,
            pltpu.SemaphoreType.DMA,
            pltpu.SemaphoreType.DMA,
        ],
    )
    def gather_max(g_hbm, idx_hbm, m_hbm, idx_v, rows_v, out_v,
                   gsem0, gsem1, ssem0, ssem1):
        wid = lax.axis_index("s") * ncores + lax.axis_index("c")
        base = wid * rows_per_w
        pltpu.sync_copy(idx_hbm.at[pl.ds(base * _K, rows_per_w * _K)], idx_v)

        def start_gather(ch, buf, sem):
            pltpu.async_copy(
                g_hbm.at[idx_v.at[pl.ds(ch * (r_chunk * _K), r_chunk * _K)]],
                rows_v.at[buf], sem)

        def wait_gather(buf, sem):
            pltpu.make_async_copy(g_hbm.at[pl.ds(0, r_chunk * _K)],
                                  rows_v.at[buf], sem).wait()

        def compute_store(ch, buf, sem):
            nch = o // nlane
            for r in range(r_chunk):
                accs = [rows_v[buf, r * _K, pl.ds(cc * nlane, nlane)]
                        for cc in range(nch)]
                for t in range(1, _K):
                    for cc in range(nch):
                        accs[cc] = jnp.maximum(
                            accs[cc],
                            rows_v[buf, r * _K + t, pl.ds(cc * nlane, nlane)])
                for cc in range(nch):
                    out_v[buf, r, pl.ds(cc * nlane, nlane)] = accs[cc]
            pltpu.async_copy(out_v.at[buf],
                             m_hbm.at[pl.ds(base + ch * r_chunk, r_chunk)],
                             sem)

        def wait_store(buf, sem):
            pltpu.make_async_copy(out_v.at[buf],
                                  m_hbm.at[pl.ds(base, r_chunk)], sem).wait()

        n_half = n_chunks // 2
        start_gather(0, 0, gsem0)

        @pl.loop(0, n_half)
        def _(p):
            ch0 = p * 2

            start_gather(ch0 + 1, 1, gsem1)
            wait_gather(0, gsem0)

            @pl.when(p > 0)
            def _():
                wait_store(0, ssem0)

            compute_store(ch0, 0, ssem0)

            @pl.when(p < n_half - 1)
            def _():
                start_gather(ch0 + 2, 0, gsem0)

            wait_gather(1, gsem1)

            @pl.when(p > 0)
            def _():
                wait_store(1, ssem1)

            compute_store(ch0 + 1, 1, ssem1)

        wait_store(0, ssem0)
        wait_store(1, ssem1)

    return gather_max


def kernel(dense_fea, W, b):
    bs, emb, n_stk, n_stk_pnt = dense_fea.shape
    n = n_stk * n_stk_pnt
    o = W.shape[0]
    x = dense_fea.reshape(bs, emb, n)
    idx, g, c = _prep_call(x, W, b.reshape(o, 1))
    idx_flat = idx[:, :, :_K].reshape(bs * n * _K)
    g_flat = g.reshape(bs * n, o)
    m = _gather_max_call(bs * n, o)(g_flat, idx_flat)
    out = _epi_call(m.reshape(bs, n, o), c)
    return out.reshape(bs, o, n_stk, n_stk_pnt)


# speedup vs baseline: 9.9375x; 1.0850x over previous
import dataclasses
import functools

import jax
import jax.numpy as jnp
from jax import lax
from jax.experimental import pallas as pl
from jax.experimental.pallas import tpu as pltpu
from jax.experimental.pallas import tpu_sc as plsc

_K = 10
_KPAD = 16


def _prep_body(x_ref, w_ref, b_ref, idx_ref, g_ref, c_ref):
    x = x_ref[0]
    d = x.shape[0]
    n = x.shape[1]
    w1 = w_ref[:, :d]
    w2 = w_ref[:, d:]

    xx = lax.dot_general(x, x, (((0,), (0,)), ((), ())),
                         preferred_element_type=jnp.float32)
    sq_row = jnp.sum(x * x, axis=0, keepdims=True)
    xt = jnp.transpose(x, (1, 0))
    sq_col = jnp.sum(xt * xt, axis=1, keepdims=True)
    m = -(sq_col - 2.0 * xx + sq_row)

    lane = lax.broadcasted_iota(jnp.int32, (n, n), 1)
    lane16 = lax.broadcasted_iota(jnp.int32, (n, _KPAD), 1)
    base = pl.program_id(0) * n
    neg = jnp.float32(-jnp.inf)
    big = jnp.int32(2**30)
    idxacc = jnp.zeros((n, _KPAD), jnp.int32)
    for t in range(_K):
        vmax = jnp.max(m, axis=1, keepdims=True)
        cand = jnp.where(m >= vmax, lane, big)
        amin = jnp.min(cand, axis=1, keepdims=True)
        idxacc = jnp.where(lane16 == t, amin + base, idxacc)
        m = jnp.where(lane == amin, neg, m)
    idx_ref[0] = idxacc

    g_ref[0] = lax.dot_general(x, w2, (((0,), (1,)), ((), ())),
                               preferred_element_type=jnp.float32,
                               precision=lax.Precision.HIGHEST
                               ).astype(jnp.bfloat16)
    c_ref[0] = lax.dot_general(w1 - w2, x, (((1,), (0,)), ((), ())),
                               preferred_element_type=jnp.float32,
                               precision=lax.Precision.HIGHEST) + b_ref[...]


def _epi_body(m_ref, c_ref, o_ref):
    z = c_ref[0] + jnp.transpose(m_ref[0].astype(jnp.float32), (1, 0))
    o_ref[0] = jnp.maximum(z, 0.2 * z)


def _prep_call(x, w, b2):
    bs, d, n = x.shape
    o = w.shape[0]
    return pl.pallas_call(
        _prep_body,
        grid=(bs,),
        in_specs=[
            pl.BlockSpec((1, d, n), lambda i: (i, 0, 0)),
            pl.BlockSpec((o, 2 * d), lambda i: (0, 0)),
            pl.BlockSpec((o, 1), lambda i: (0, 0)),
        ],
        out_specs=[
            pl.BlockSpec((1, n, _KPAD), lambda i: (i, 0, 0)),
            pl.BlockSpec((1, n, o), lambda i: (i, 0, 0)),
            pl.BlockSpec((1, o, n), lambda i: (i, 0, 0)),
        ],
        out_shape=[
            jax.ShapeDtypeStruct((bs, n, _KPAD), jnp.int32),
            jax.ShapeDtypeStruct((bs, n, o), jnp.bfloat16),
            jax.ShapeDtypeStruct((bs, o, n), jnp.float32),
        ],
    )(x, w, b2)


def _epi_call(m, c):
    bs, n, o = m.shape
    return pl.pallas_call(
        _epi_body,
        grid=(bs,),
        in_specs=[
            pl.BlockSpec((1, n, o), lambda i: (i, 0, 0)),
            pl.BlockSpec((1, o, n), lambda i: (i, 0, 0)),
        ],
        out_specs=pl.BlockSpec((1, o, n), lambda i: (i, 0, 0)),
        out_shape=jax.ShapeDtypeStruct((bs, o, n), jnp.float32),
    )(m, c)


@functools.lru_cache(maxsize=None)
def _gather_max_call(rows, o):
    ncores, nsub = 2, 16
    nw = ncores * nsub
    rows_per_w = rows // nw
    r_chunk = 8
    n_chunks = rows_per_w // r_chunk
    nlane = 16
    mesh = plsc.VectorSubcoreMesh(core_axis_name="c", subcore_axis_name="s")

    cp = pltpu.CompilerParams()
    if "needs_layout_passes" in pltpu.CompilerParams.__dataclass_fields__:
        cp = dataclasses.replace(cp, needs_layout_passes=False)

    @functools.partial(
        pl.kernel,
        mesh=mesh,
        compiler_params=cp,
        out_type=jax.ShapeDtypeStruct((rows, o), jnp.int32),
        scratch_types=[
            pltpu.VMEM((rows_per_w * _K,), jnp.int32),
            pltpu.VMEM((2, r_chunk * _K, o), jnp.int32),
            pltpu.VMEM((2, r_chunk, o), jnp.int32),
            pltpu.SemaphoreType.DMA,
            pltpu.SemaphoreType.DMA,
            pltpu.SemaphoreType.DMA,
            pltpu.SemaphoreType.DMA,
        ],
    )
    def gather_max(g_hbm, idx_hbm, m_hbm, idx_v, rows_v, out_v,
                   gsem0, gsem1, ssem0, ssem1):
        wid = lax.axis_index("s") * ncores + lax.axis_index("c")
        base = wid * rows_per_w
        pltpu.sync_copy(idx_hbm.at[pl.ds(base * _K, rows_per_w * _K)], idx_v)

        def start_gather(ch, buf, sem):
            pltpu.async_copy(
                g_hbm.at[idx_v.at[pl.ds(ch * (r_chunk * _K), r_chunk * _K)]],
                rows_v.at[buf], sem)

        def wait_gather(buf, sem):
            pltpu.make_async_copy(g_hbm.at[pl.ds(0, r_chunk * _K)],
                                  rows_v.at[buf], sem).wait()

        def compute_store(ch, buf, sem):
            nch = o // nlane
            for r in range(r_chunk):
                accs = [plsc.bitcast(
                            rows_v[buf, r * _K, pl.ds(cc * nlane, nlane)],
                            jnp.bfloat16)
                        for cc in range(nch)]
                for t in range(1, _K):
                    for cc in range(nch):
                        accs[cc] = jnp.maximum(
                            accs[cc],
                            plsc.bitcast(
                                rows_v[buf, r * _K + t,
                                       pl.ds(cc * nlane, nlane)],
                                jnp.bfloat16))
                for cc in range(nch):
                    out_v[buf, r, pl.ds(cc * nlane, nlane)] = plsc.bitcast(
                        accs[cc], jnp.int32)
            pltpu.async_copy(out_v.at[buf],
                             m_hbm.at[pl.ds(base + ch * r_chunk, r_chunk)],
                             sem)

        def wait_store(buf, sem):
            pltpu.make_async_copy(out_v.at[buf],
                                  m_hbm.at[pl.ds(base, r_chunk)], sem).wait()

        n_half = n_chunks // 2
        start_gather(0, 0, gsem0)

        @pl.loop(0, n_half)
        def _(p):
            ch0 = p * 2

            start_gather(ch0 + 1, 1, gsem1)
            wait_gather(0, gsem0)

            @pl.when(p > 0)
            def _():
                wait_store(0, ssem0)

            compute_store(ch0, 0, ssem0)

            @pl.when(p < n_half - 1)
            def _():
                start_gather(ch0 + 2, 0, gsem0)

            wait_gather(1, gsem1)

            @pl.when(p > 0)
            def _():
                wait_store(1, ssem1)

            compute_store(ch0 + 1, 1, ssem1)

        wait_store(0, ssem0)
        wait_store(1, ssem1)

    return gather_max


def kernel(dense_fea, W, b):
    bs, emb, n_stk, n_stk_pnt = dense_fea.shape
    n = n_stk * n_stk_pnt
    o = W.shape[0]
    x = dense_fea.reshape(bs, emb, n)
    idx, g, c = _prep_call(x, W, b.reshape(o, 1))
    idx_flat = idx[:, :, :_K].reshape(bs * n * _K)
    g_pack = lax.bitcast_convert_type(
        g.reshape(bs * n, o // 2, 2), jnp.int32)
    m_pack = _gather_max_call(bs * n, o // 2)(g_pack, idx_flat)
    m = lax.bitcast_convert_type(m_pack, jnp.bfloat16).reshape(bs, n, o)
    out = _epi_call(m, c)
    return out.reshape(bs, o, n_stk, n_stk_pnt)


# speedup vs baseline: 10.5585x; 1.0625x over previous
import dataclasses
import functools

import jax
import jax.numpy as jnp
from jax import lax
from jax.experimental import pallas as pl
from jax.experimental.pallas import tpu as pltpu
from jax.experimental.pallas import tpu_sc as plsc

_K = 10
_KPAD = 16


def _prep_body(x_ref, w_ref, b_ref, idx_ref, g_ref, c_ref):
    x = x_ref[0]
    d = x.shape[0]
    n = x.shape[1]
    w1 = w_ref[:, :d]
    w2 = w_ref[:, d:]

    xx = lax.dot_general(x, x, (((0,), (0,)), ((), ())),
                         preferred_element_type=jnp.float32)
    sq_row = jnp.sum(x * x, axis=0, keepdims=True)
    xt = jnp.transpose(x, (1, 0))
    sq_col = jnp.sum(xt * xt, axis=1, keepdims=True)
    m = -(sq_col - 2.0 * xx + sq_row)

    lane = lax.broadcasted_iota(jnp.int32, (n, n), 1)
    lane16 = lax.broadcasted_iota(jnp.int32, (n, _KPAD), 1)
    base = pl.program_id(0) * n
    neg = jnp.float32(-jnp.inf)
    big = jnp.int32(2**30)
    idxacc = jnp.zeros((n, _KPAD), jnp.int32)
    for t in range(_K):
        vmax = jnp.max(m, axis=1, keepdims=True)
        cand = jnp.where(m >= vmax, lane, big)
        amin = jnp.min(cand, axis=1, keepdims=True)
        idxacc = jnp.where(lane16 == t, amin + base, idxacc)
        m = jnp.where(lane == amin, neg, m)
    idx_ref[0] = idxacc

    g_ref[0] = lax.dot_general(x, w2, (((0,), (1,)), ((), ())),
                               preferred_element_type=jnp.float32,
                               precision=lax.Precision.HIGHEST
                               ).astype(jnp.bfloat16)
    c_ref[0] = lax.dot_general(w1 - w2, x, (((1,), (0,)), ((), ())),
                               preferred_element_type=jnp.float32,
                               precision=lax.Precision.HIGHEST) + b_ref[...]


def _epi_body(m_ref, c_ref, o_ref):
    z = c_ref[0] + jnp.transpose(m_ref[0].astype(jnp.float32), (1, 0))
    o_ref[0] = jnp.maximum(z, 0.2 * z)


def _prep_call(x, w, b2):
    bs, d, n = x.shape
    o = w.shape[0]
    return pl.pallas_call(
        _prep_body,
        grid=(bs,),
        in_specs=[
            pl.BlockSpec((1, d, n), lambda i: (i, 0, 0)),
            pl.BlockSpec((o, 2 * d), lambda i: (0, 0)),
            pl.BlockSpec((o, 1), lambda i: (0, 0)),
        ],
        out_specs=[
            pl.BlockSpec((1, n, _KPAD), lambda i: (i, 0, 0)),
            pl.BlockSpec((1, n, o), lambda i: (i, 0, 0)),
            pl.BlockSpec((1, o, n), lambda i: (i, 0, 0)),
        ],
        out_shape=[
            jax.ShapeDtypeStruct((bs, n, _KPAD), jnp.int32),
            jax.ShapeDtypeStruct((bs, n, o), jnp.bfloat16),
            jax.ShapeDtypeStruct((bs, o, n), jnp.float32),
        ],
    )(x, w, b2)


def _epi_call(m, c):
    bs, n, o = m.shape
    return pl.pallas_call(
        _epi_body,
        grid=(bs,),
        in_specs=[
            pl.BlockSpec((1, n, o), lambda i: (i, 0, 0)),
            pl.BlockSpec((1, o, n), lambda i: (i, 0, 0)),
        ],
        out_specs=pl.BlockSpec((1, o, n), lambda i: (i, 0, 0)),
        out_shape=jax.ShapeDtypeStruct((bs, o, n), jnp.float32),
    )(m, c)


@functools.lru_cache(maxsize=None)
def _gather_max_call(rows, o):
    ncores, nsub = 2, 16
    nw = ncores * nsub
    rows_per_w = rows // nw
    r_chunk = 8
    n_chunks = rows_per_w // r_chunk
    nlane = 16
    mesh = plsc.VectorSubcoreMesh(core_axis_name="c", subcore_axis_name="s")

    cp = pltpu.CompilerParams()
    if "needs_layout_passes" in pltpu.CompilerParams.__dataclass_fields__:
        cp = dataclasses.replace(cp, needs_layout_passes=False)

    @functools.partial(
        pl.kernel,
        mesh=mesh,
        compiler_params=cp,
        out_type=jax.ShapeDtypeStruct((rows, o), jnp.int32),
        scratch_types=[
            pltpu.VMEM((rows_per_w * _K,), jnp.int32),
            pltpu.VMEM((2, r_chunk * _K, o), jnp.int32),
            pltpu.VMEM((2, r_chunk, o), jnp.int32),
            pltpu.SemaphoreType.DMA,
            pltpu.SemaphoreType.DMA,
            pltpu.SemaphoreType.DMA,
            pltpu.SemaphoreType.DMA,
        ],
    )
    def gather_max(g_hbm, idx_hbm, m_hbm, idx_v, rows_v, out_v,
                   gsem0, gsem1, ssem0, ssem1):
        wid = lax.axis_index("s") * ncores + lax.axis_index("c")
        base = wid * rows_per_w
        pltpu.sync_copy(idx_hbm.at[pl.ds(base * _K, rows_per_w * _K)], idx_v)

        def start_gather(ch, buf, sem):
            pltpu.async_copy(
                g_hbm.at[idx_v.at[pl.ds(ch * (r_chunk * _K), r_chunk * _K)]],
                rows_v.at[buf], sem)

        def wait_gather(buf, sem):
            pltpu.make_async_copy(g_hbm.at[pl.ds(0, r_chunk * _K)],
                                  rows_v.at[buf], sem).wait()

        def compute_store(ch, buf, sem):
            nch = o // nlane
            for r in range(r_chunk):
                accs = [plsc.bitcast(
                            rows_v[buf, r * _K, pl.ds(cc * nlane, nlane)],
                            jnp.bfloat16)
                        for cc in range(nch)]
                for t in range(1, _K):
                    for cc in range(nch):
                        accs[cc] = jnp.maximum(
                            accs[cc],
                            plsc.bitcast(
                                rows_v[buf, r * _K + t,
                                       pl.ds(cc * nlane, nlane)],
                                jnp.bfloat16))
                for cc in range(nch):
                    out_v[buf, r, pl.ds(cc * nlane, nlane)] = plsc.bitcast(
                        accs[cc], jnp.int32)
            pltpu.async_copy(out_v.at[buf],
                             m_hbm.at[pl.ds(base + ch * r_chunk, r_chunk)],
                             sem)

        def wait_store(buf, sem):
            pltpu.make_async_copy(out_v.at[buf],
                                  m_hbm.at[pl.ds(base, r_chunk)], sem).wait()

        n_half = n_chunks // 2
        start_gather(0, 0, gsem0)

        @pl.loop(0, n_half)
        def _(p):
            ch0 = p * 2

            start_gather(ch0 + 1, 1, gsem1)
            wait_gather(0, gsem0)

            @pl.when(p > 0)
            def _():
                wait_store(0, ssem0)

            compute_store(ch0, 0, ssem0)

            @pl.when(p < n_half - 1)
            def _():
                start_gather(ch0 + 2, 0, gsem0)

            wait_gather(1, gsem1)

            @pl.when(p > 0)
            def _():
                wait_store(1, ssem1)

            compute_store(ch0 + 1, 1, ssem1)

        wait_store(0, ssem0)
        wait_store(1, ssem1)

    return gather_max


def kernel(dense_fea, W, b):
    bs, emb, n_stk, n_stk_pnt = dense_fea.shape
    n = n_stk * n_stk_pnt
    o = W.shape[0]
    x = dense_fea.reshape(bs, emb, n)
    b2 = b.reshape(o, 1)
    splits = 4
    bsz = bs // splits
    parts = []
    for s in range(splits):
        xs = x[s * bsz:(s + 1) * bsz]
        idx, g, c = _prep_call(xs, W, b2)
        idx_flat = idx[:, :, :_K].reshape(bsz * n * _K)
        g_pack = lax.bitcast_convert_type(
            g.reshape(bsz * n, o // 2, 2), jnp.int32)
        m_pack = _gather_max_call(bsz * n, o // 2)(g_pack, idx_flat)
        m = lax.bitcast_convert_type(m_pack, jnp.bfloat16).reshape(bsz, n, o)
        parts.append(_epi_call(m, c))
    out = jnp.concatenate(parts, axis=0)
    return out.reshape(bs, o, n_stk, n_stk_pnt)


# speedup vs baseline: 10.5909x; 1.0031x over previous
import dataclasses
import functools

import jax
import jax.numpy as jnp
from jax import lax
from jax.experimental import pallas as pl
from jax.experimental.pallas import tpu as pltpu
from jax.experimental.pallas import tpu_sc as plsc

_K = 10
_KPAD = 16


def _prep_body(x_ref, w_ref, b_ref, idx_ref, g_ref, c_ref):
    x = x_ref[0]
    d = x.shape[0]
    n = x.shape[1]
    w1 = w_ref[:, :d]
    w2 = w_ref[:, d:]

    xx = lax.dot_general(x, x, (((0,), (0,)), ((), ())),
                         preferred_element_type=jnp.float32)
    sq_row = jnp.sum(x * x, axis=0, keepdims=True)
    xt = jnp.transpose(x, (1, 0))
    sq_col = jnp.sum(xt * xt, axis=1, keepdims=True)
    m = -(sq_col - 2.0 * xx + sq_row)

    lane = lax.broadcasted_iota(jnp.int32, (n, n), 1)
    lane16 = lax.broadcasted_iota(jnp.int32, (n, _KPAD), 1)
    base = pl.program_id(0) * n
    neg = jnp.float32(-jnp.inf)
    big = jnp.int32(2**30)
    idxacc = jnp.zeros((n, _KPAD), jnp.int32)
    for t in range(_K):
        vmax = jnp.max(m, axis=1, keepdims=True)
        cand = jnp.where(m >= vmax, lane, big)
        amin = jnp.min(cand, axis=1, keepdims=True)
        idxacc = jnp.where(lane16 == t, amin + base, idxacc)
        m = jnp.where(lane == amin, neg, m)
    idx_ref[0] = idxacc

    g_ref[0] = lax.dot_general(x, w2, (((0,), (1,)), ((), ())),
                               preferred_element_type=jnp.float32,
                               precision=lax.Precision.HIGHEST
                               ).astype(jnp.bfloat16)
    c_ref[0] = lax.dot_general(w1 - w2, x, (((1,), (0,)), ((), ())),
                               preferred_element_type=jnp.float32,
                               precision=lax.Precision.HIGHEST) + b_ref[...]


def _epi_body(m_ref, c_ref, o_ref):
    z = c_ref[0] + jnp.transpose(m_ref[0].astype(jnp.float32), (1, 0))
    o_ref[0] = jnp.maximum(z, 0.2 * z)


def _prep_call(x, w, b2):
    bs, d, n = x.shape
    o = w.shape[0]
    return pl.pallas_call(
        _prep_body,
        grid=(bs,),
        compiler_params=pltpu.CompilerParams(
            dimension_semantics=("parallel",)),
        in_specs=[
            pl.BlockSpec((1, d, n), lambda i: (i, 0, 0)),
            pl.BlockSpec((o, 2 * d), lambda i: (0, 0)),
            pl.BlockSpec((o, 1), lambda i: (0, 0)),
        ],
        out_specs=[
            pl.BlockSpec((1, n, _KPAD), lambda i: (i, 0, 0)),
            pl.BlockSpec((1, n, o), lambda i: (i, 0, 0)),
            pl.BlockSpec((1, o, n), lambda i: (i, 0, 0)),
        ],
        out_shape=[
            jax.ShapeDtypeStruct((bs, n, _KPAD), jnp.int32),
            jax.ShapeDtypeStruct((bs, n, o), jnp.bfloat16),
            jax.ShapeDtypeStruct((bs, o, n), jnp.float32),
        ],
    )(x, w, b2)


def _epi_call(m, c):
    bs, n, o = m.shape
    return pl.pallas_call(
        _epi_body,
        grid=(bs,),
        compiler_params=pltpu.CompilerParams(
            dimension_semantics=("parallel",)),
        in_specs=[
            pl.BlockSpec((1, n, o), lambda i: (i, 0, 0)),
            pl.BlockSpec((1, o, n), lambda i: (i, 0, 0)),
        ],
        out_specs=pl.BlockSpec((1, o, n), lambda i: (i, 0, 0)),
        out_shape=jax.ShapeDtypeStruct((bs, o, n), jnp.float32),
    )(m, c)


@functools.lru_cache(maxsize=None)
def _gather_max_call(rows, o):
    ncores, nsub = 2, 16
    nw = ncores * nsub
    rows_per_w = rows // nw
    r_chunk = 8
    n_chunks = rows_per_w // r_chunk
    nlane = 16
    mesh = plsc.VectorSubcoreMesh(core_axis_name="c", subcore_axis_name="s")

    cp = pltpu.CompilerParams()
    if "needs_layout_passes" in pltpu.CompilerParams.__dataclass_fields__:
        cp = dataclasses.replace(cp, needs_layout_passes=False)

    @functools.partial(
        pl.kernel,
        mesh=mesh,
        compiler_params=cp,
        out_type=jax.ShapeDtypeStruct((rows, o), jnp.int32),
        scratch_types=[
            pltpu.VMEM((rows_per_w * _K,), jnp.int32),
            pltpu.VMEM((2, r_chunk * _K, o), jnp.int32),
            pltpu.VMEM((2, r_chunk, o), jnp.int32),
            pltpu.SemaphoreType.DMA,
            pltpu.SemaphoreType.DMA,
            pltpu.SemaphoreType.DMA,
            pltpu.SemaphoreType.DMA,
        ],
    )
    def gather_max(g_hbm, idx_hbm, m_hbm, idx_v, rows_v, out_v,
                   gsem0, gsem1, ssem0, ssem1):
        wid = lax.axis_index("s") * ncores + lax.axis_index("c")
        base = wid * rows_per_w
        pltpu.sync_copy(idx_hbm.at[pl.ds(base * _K, rows_per_w * _K)], idx_v)

        def start_gather(ch, buf, sem):
            pltpu.async_copy(
                g_hbm.at[idx_v.at[pl.ds(ch * (r_chunk * _K), r_chunk * _K)]],
                rows_v.at[buf], sem)

        def wait_gather(buf, sem):
            pltpu.make_async_copy(g_hbm.at[pl.ds(0, r_chunk * _K)],
                                  rows_v.at[buf], sem).wait()

        def compute_store(ch, buf, sem):
            nch = o // nlane
            for r in range(r_chunk):
                accs = [plsc.bitcast(
                            rows_v[buf, r * _K, pl.ds(cc * nlane, nlane)],
                            jnp.bfloat16)
                        for cc in range(nch)]
                for t in range(1, _K):
                    for cc in range(nch):
                        accs[cc] = jnp.maximum(
                            accs[cc],
                            plsc.bitcast(
                                rows_v[buf, r * _K + t,
                                       pl.ds(cc * nlane, nlane)],
                                jnp.bfloat16))
                for cc in range(nch):
                    out_v[buf, r, pl.ds(cc * nlane, nlane)] = plsc.bitcast(
                        accs[cc], jnp.int32)
            pltpu.async_copy(out_v.at[buf],
                             m_hbm.at[pl.ds(base + ch * r_chunk, r_chunk)],
                             sem)

        def wait_store(buf, sem):
            pltpu.make_async_copy(out_v.at[buf],
                                  m_hbm.at[pl.ds(base, r_chunk)], sem).wait()

        n_half = n_chunks // 2
        start_gather(0, 0, gsem0)

        @pl.loop(0, n_half)
        def _(p):
            ch0 = p * 2

            start_gather(ch0 + 1, 1, gsem1)
            wait_gather(0, gsem0)

            @pl.when(p > 0)
            def _():
                wait_store(0, ssem0)

            compute_store(ch0, 0, ssem0)

            @pl.when(p < n_half - 1)
            def _():
                start_gather(ch0 + 2, 0, gsem0)

            wait_gather(1, gsem1)

            @pl.when(p > 0)
            def _():
                wait_store(1, ssem1)

            compute_store(ch0 + 1, 1, ssem1)

        wait_store(0, ssem0)
        wait_store(1, ssem1)

    return gather_max


def kernel(dense_fea, W, b):
    bs, emb, n_stk, n_stk_pnt = dense_fea.shape
    n = n_stk * n_stk_pnt
    o = W.shape[0]
    x = dense_fea.reshape(bs, emb, n)
    b2 = b.reshape(o, 1)
    splits = 4
    bsz = bs // splits
    parts = []
    for s in range(splits):
        xs = x[s * bsz:(s + 1) * bsz]
        idx, g, c = _prep_call(xs, W, b2)
        idx_flat = idx[:, :, :_K].reshape(bsz * n * _K)
        g_pack = lax.bitcast_convert_type(
            g.reshape(bsz * n, o // 2, 2), jnp.int32)
        m_pack = _gather_max_call(bsz * n, o // 2)(g_pack, idx_flat)
        m = lax.bitcast_convert_type(m_pack, jnp.bfloat16).reshape(bsz, n, o)
        parts.append(_epi_call(m, c))
    out = jnp.concatenate(parts, axis=0)
    return out.reshape(bs, o, n_stk, n_stk_pnt)


# speedup vs baseline: 10.8191x; 1.0215x over previous
import dataclasses
import functools

import jax
import jax.numpy as jnp
from jax import lax
from jax.experimental import pallas as pl
from jax.experimental.pallas import tpu as pltpu
from jax.experimental.pallas import tpu_sc as plsc

_K = 10
_KPAD = 16


def _prep_body(x_ref, w_ref, b_ref, idx_ref, g_ref, c_ref):
    x = x_ref[0]
    d = x.shape[0]
    n = x.shape[1]
    w1 = w_ref[:, :d]
    w2 = w_ref[:, d:]

    xx = lax.dot_general(x, x, (((0,), (0,)), ((), ())),
                         preferred_element_type=jnp.float32)
    sq_row = jnp.sum(x * x, axis=0, keepdims=True)
    xt = jnp.transpose(x, (1, 0))
    sq_col = jnp.sum(xt * xt, axis=1, keepdims=True)
    m = -(sq_col - 2.0 * xx + sq_row)

    lane = lax.broadcasted_iota(jnp.int32, (n, n), 1)
    lane16 = lax.broadcasted_iota(jnp.int32, (n, _KPAD), 1)
    base = pl.program_id(0) * n
    neg = jnp.float32(-jnp.inf)
    big = jnp.int32(2**30)
    idxacc = jnp.zeros((n, _KPAD), jnp.int32)
    for t in range(_K):
        vmax = jnp.max(m, axis=1, keepdims=True)
        cand = jnp.where(m >= vmax, lane, big)
        amin = jnp.min(cand, axis=1, keepdims=True)
        idxacc = jnp.where(lane16 == t, amin + base, idxacc)
        m = jnp.where(lane == amin, neg, m)
    idx_ref[0] = idxacc

    g_ref[0] = lax.dot_general(x, w2, (((0,), (1,)), ((), ())),
                               preferred_element_type=jnp.float32,
                               precision=lax.Precision.HIGHEST
                               ).astype(jnp.bfloat16)
    c_ref[0] = lax.dot_general(w1 - w2, x, (((1,), (0,)), ((), ())),
                               preferred_element_type=jnp.float32,
                               precision=lax.Precision.HIGHEST) + b_ref[...]


def _epi_body(m_ref, c_ref, o_ref):
    z = c_ref[0] + jnp.transpose(m_ref[0].astype(jnp.float32), (1, 0))
    o_ref[0] = jnp.maximum(z, 0.2 * z)


def _prep_call(x, w, b2):
    bs, d, n = x.shape
    o = w.shape[0]
    return pl.pallas_call(
        _prep_body,
        grid=(bs,),
        compiler_params=pltpu.CompilerParams(
            dimension_semantics=("parallel",)),
        in_specs=[
            pl.BlockSpec((1, d, n), lambda i: (i, 0, 0)),
            pl.BlockSpec((o, 2 * d), lambda i: (0, 0)),
            pl.BlockSpec((o, 1), lambda i: (0, 0)),
        ],
        out_specs=[
            pl.BlockSpec((1, n, _KPAD), lambda i: (i, 0, 0)),
            pl.BlockSpec((1, n, o), lambda i: (i, 0, 0)),
            pl.BlockSpec((1, o, n), lambda i: (i, 0, 0)),
        ],
        out_shape=[
            jax.ShapeDtypeStruct((bs, n, _KPAD), jnp.int32),
            jax.ShapeDtypeStruct((bs, n, o), jnp.bfloat16),
            jax.ShapeDtypeStruct((bs, o, n), jnp.float32),
        ],
    )(x, w, b2)


def _epi_call(m, c):
    bs, n, o = m.shape
    return pl.pallas_call(
        _epi_body,
        grid=(bs,),
        compiler_params=pltpu.CompilerParams(
            dimension_semantics=("parallel",)),
        in_specs=[
            pl.BlockSpec((1, n, o), lambda i: (i, 0, 0)),
            pl.BlockSpec((1, o, n), lambda i: (i, 0, 0)),
        ],
        out_specs=pl.BlockSpec((1, o, n), lambda i: (i, 0, 0)),
        out_shape=jax.ShapeDtypeStruct((bs, o, n), jnp.float32),
    )(m, c)


@functools.lru_cache(maxsize=None)
def _gather_max_call(rows, o):
    ncores, nsub = 2, 16
    nw = ncores * nsub
    rows_per_w = rows // nw
    r_chunk = 8
    n_chunks = rows_per_w // r_chunk
    nlane = 16
    mesh = plsc.VectorSubcoreMesh(core_axis_name="c", subcore_axis_name="s")

    cp = pltpu.CompilerParams()
    if "needs_layout_passes" in pltpu.CompilerParams.__dataclass_fields__:
        cp = dataclasses.replace(cp, needs_layout_passes=False)

    @functools.partial(
        pl.kernel,
        mesh=mesh,
        compiler_params=cp,
        out_type=jax.ShapeDtypeStruct((rows, o), jnp.int32),
        scratch_types=[
            pltpu.VMEM((rows_per_w * _K,), jnp.int32),
            pltpu.VMEM((2, r_chunk * _K, o), jnp.int32),
            pltpu.VMEM((2, r_chunk, o), jnp.int32),
            pltpu.VMEM_SHARED((rows, o), jnp.int32),
            pltpu.SemaphoreType.DMA,
            pltpu.SemaphoreType.DMA,
            pltpu.SemaphoreType.DMA,
            pltpu.SemaphoreType.DMA,
        ],
    )
    def gather_max(g_hbm, idx_hbm, m_hbm, idx_v, rows_v, out_v, g_sh,
                   gsem0, gsem1, ssem0, ssem1):
        sid = lax.axis_index("s")
        wid = sid * ncores + lax.axis_index("c")
        base = wid * rows_per_w
        seg = rows // nsub
        pltpu.sync_copy(g_hbm.at[pl.ds(sid * seg, seg)],
                        g_sh.at[pl.ds(sid * seg, seg)])
        pltpu.sync_copy(idx_hbm.at[pl.ds(base * _K, rows_per_w * _K)], idx_v)
        plsc.subcore_barrier()

        def start_gather(ch, buf, sem):
            pltpu.async_copy(
                g_sh.at[idx_v.at[pl.ds(ch * (r_chunk * _K), r_chunk * _K)]],
                rows_v.at[buf], sem)

        def wait_gather(buf, sem):
            pltpu.make_async_copy(g_hbm.at[pl.ds(0, r_chunk * _K)],
                                  rows_v.at[buf], sem).wait()

        def compute_store(ch, buf, sem):
            nch = o // nlane
            for r in range(r_chunk):
                accs = [plsc.bitcast(
                            rows_v[buf, r * _K, pl.ds(cc * nlane, nlane)],
                            jnp.bfloat16)
                        for cc in range(nch)]
                for t in range(1, _K):
                    for cc in range(nch):
                        accs[cc] = jnp.maximum(
                            accs[cc],
                            plsc.bitcast(
                                rows_v[buf, r * _K + t,
                                       pl.ds(cc * nlane, nlane)],
                                jnp.bfloat16))
                for cc in range(nch):
                    out_v[buf, r, pl.ds(cc * nlane, nlane)] = plsc.bitcast(
                        accs[cc], jnp.int32)
            pltpu.async_copy(out_v.at[buf],
                             m_hbm.at[pl.ds(base + ch * r_chunk, r_chunk)],
                             sem)

        def wait_store(buf, sem):
            pltpu.make_async_copy(out_v.at[buf],
                                  m_hbm.at[pl.ds(base, r_chunk)], sem).wait()

        n_half = n_chunks // 2
        start_gather(0, 0, gsem0)

        @pl.loop(0, n_half)
        def _(p):
            ch0 = p * 2

            start_gather(ch0 + 1, 1, gsem1)
            wait_gather(0, gsem0)

            @pl.when(p > 0)
            def _():
                wait_store(0, ssem0)

            compute_store(ch0, 0, ssem0)

            @pl.when(p < n_half - 1)
            def _():
                start_gather(ch0 + 2, 0, gsem0)

            wait_gather(1, gsem1)

            @pl.when(p > 0)
            def _():
                wait_store(1, ssem1)

            compute_store(ch0 + 1, 1, ssem1)

        wait_store(0, ssem0)
        wait_store(1, ssem1)

    return gather_max


def kernel(dense_fea, W, b):
    bs, emb, n_stk, n_stk_pnt = dense_fea.shape
    n = n_stk * n_stk_pnt
    o = W.shape[0]
    x = dense_fea.reshape(bs, emb, n)
    b2 = b.reshape(o, 1)
    splits = 4
    bsz = bs // splits
    parts = []
    for s in range(splits):
        xs = x[s * bsz:(s + 1) * bsz]
        idx, g, c = _prep_call(xs, W, b2)
        idx_flat = idx[:, :, :_K].reshape(bsz * n * _K)
        g_pack = lax.bitcast_convert_type(
            g.reshape(bsz * n, o // 2, 2), jnp.int32)
        m_pack = _gather_max_call(bsz * n, o // 2)(g_pack, idx_flat)
        m = lax.bitcast_convert_type(m_pack, jnp.bfloat16).reshape(bsz, n, o)
        parts.append(_epi_call(m, c))
    out = jnp.concatenate(parts, axis=0)
    return out.reshape(bs, o, n_stk, n_stk_pnt)


# speedup vs baseline: 15.1807x; 1.4031x over previous
import dataclasses
import functools

import jax
import jax.numpy as jnp
from jax import lax
from jax.experimental import pallas as pl
from jax.experimental.pallas import tpu as pltpu
from jax.experimental.pallas import tpu_sc as plsc

_K = 10
_KPAD = 16


def _prep_body(x_ref, w_ref, b_ref, idx_ref, g_ref, c_ref):
    x = x_ref[0]
    d = x.shape[0]
    n = x.shape[1]
    w1 = w_ref[:, :d]
    w2 = w_ref[:, d:]

    xx = lax.dot_general(x, x, (((0,), (0,)), ((), ())),
                         preferred_element_type=jnp.float32)
    sq_row = jnp.sum(x * x, axis=0, keepdims=True)
    xt = jnp.transpose(x, (1, 0))
    sq_col = jnp.sum(xt * xt, axis=1, keepdims=True)
    m = -(sq_col - 2.0 * xx + sq_row)

    lane = lax.broadcasted_iota(jnp.int32, (n, n), 1)
    lane16 = lax.broadcasted_iota(jnp.int32, (n, _KPAD), 1)
    base = pl.program_id(0) * n
    neg = jnp.float32(-jnp.inf)
    big = jnp.int32(2**30)
    idxacc = jnp.zeros((n, _KPAD), jnp.int32)
    for t in range(_K):
        vmax = jnp.max(m, axis=1, keepdims=True)
        cand = jnp.where(m >= vmax, lane, big)
        amin = jnp.min(cand, axis=1, keepdims=True)
        idxacc = jnp.where(lane16 == t, amin + base, idxacc)
        m = jnp.where(lane == amin, neg, m)
    idx_ref[0] = idxacc

    g32 = lax.dot_general(x, w2, (((0,), (1,)), ((), ())),
                          preferred_element_type=jnp.float32,
                          precision=lax.Precision.HIGHEST)
    u = lax.bitcast_convert_type(g32, jnp.uint32)
    r = (u + jnp.uint32(0x7FFF) + ((u >> 16) & jnp.uint32(1))) >> 16
    half = r.shape[1] // 2
    gw = r[:, :half] | (r[:, half:] << 16)
    g_ref[0] = lax.bitcast_convert_type(gw, jnp.int32)
    c_ref[0] = lax.dot_general(w1 - w2, x, (((1,), (0,)), ((), ())),
                               preferred_element_type=jnp.float32,
                               precision=lax.Precision.HIGHEST) + b_ref[...]


def _epi_body(m_ref, c_ref, o_ref):
    u = lax.bitcast_convert_type(m_ref[0], jnp.uint32)
    lo = lax.bitcast_convert_type(u << 16, jnp.float32)
    hi = lax.bitcast_convert_type(u & jnp.uint32(0xFFFF0000), jnp.float32)
    mt = jnp.concatenate([lo, hi], axis=1)
    z = c_ref[0] + jnp.transpose(mt, (1, 0))
    o_ref[0] = jnp.maximum(z, 0.2 * z)


def _prep_call(x, w, b2):
    bs, d, n = x.shape
    o = w.shape[0]
    return pl.pallas_call(
        _prep_body,
        grid=(bs,),
        compiler_params=pltpu.CompilerParams(
            dimension_semantics=("parallel",)),
        in_specs=[
            pl.BlockSpec((1, d, n), lambda i: (i, 0, 0)),
            pl.BlockSpec((o, 2 * d), lambda i: (0, 0)),
            pl.BlockSpec((o, 1), lambda i: (0, 0)),
        ],
        out_specs=[
            pl.BlockSpec((1, n, _KPAD), lambda i: (i, 0, 0)),
            pl.BlockSpec((1, n, o // 2), lambda i: (i, 0, 0)),
            pl.BlockSpec((1, o, n), lambda i: (i, 0, 0)),
        ],
        out_shape=[
            jax.ShapeDtypeStruct((bs, n, _KPAD), jnp.int32),
            jax.ShapeDtypeStruct((bs, n, o // 2), jnp.int32),
            jax.ShapeDtypeStruct((bs, o, n), jnp.float32),
        ],
    )(x, w, b2)


def _epi_call(m, c):
    bs, n, ow = m.shape
    o = 2 * ow
    return pl.pallas_call(
        _epi_body,
        grid=(bs,),
        compiler_params=pltpu.CompilerParams(
            dimension_semantics=("parallel",)),
        in_specs=[
            pl.BlockSpec((1, n, ow), lambda i: (i, 0, 0)),
            pl.BlockSpec((1, o, n), lambda i: (i, 0, 0)),
        ],
        out_specs=pl.BlockSpec((1, o, n), lambda i: (i, 0, 0)),
        out_shape=jax.ShapeDtypeStruct((bs, o, n), jnp.float32),
    )(m, c)


@functools.lru_cache(maxsize=None)
def _gather_max_call(rows, o):
    ncores, nsub = 2, 16
    nw = ncores * nsub
    rows_per_w = rows // nw
    r_chunk = 8
    n_chunks = rows_per_w // r_chunk
    nlane = 16
    mesh = plsc.VectorSubcoreMesh(core_axis_name="c", subcore_axis_name="s")

    cp = pltpu.CompilerParams()
    if "needs_layout_passes" in pltpu.CompilerParams.__dataclass_fields__:
        cp = dataclasses.replace(cp, needs_layout_passes=False)

    @functools.partial(
        pl.kernel,
        mesh=mesh,
        compiler_params=cp,
        out_type=jax.ShapeDtypeStruct((rows, o), jnp.int32),
        scratch_types=[
            pltpu.VMEM((rows_per_w * _K,), jnp.int32),
            pltpu.VMEM((2, r_chunk * _K, o), jnp.int32),
            pltpu.VMEM((2, r_chunk, o), jnp.int32),
            pltpu.VMEM_SHARED((rows, o), jnp.int32),
            pltpu.SemaphoreType.DMA,
            pltpu.SemaphoreType.DMA,
            pltpu.SemaphoreType.DMA,
            pltpu.SemaphoreType.DMA,
        ],
    )
    def gather_max(g_hbm, idx_hbm, m_hbm, idx_v, rows_v, out_v, g_sh,
                   gsem0, gsem1, ssem0, ssem1):
        sid = lax.axis_index("s")
        wid = sid * ncores + lax.axis_index("c")
        base = wid * rows_per_w
        seg = rows // nsub
        pltpu.sync_copy(g_hbm.at[pl.ds(sid * seg, seg)],
                        g_sh.at[pl.ds(sid * seg, seg)])
        pltpu.sync_copy(idx_hbm.at[pl.ds(base * _K, rows_per_w * _K)], idx_v)
        plsc.subcore_barrier()

        def start_gather(ch, buf, sem):
            pltpu.async_copy(
                g_sh.at[idx_v.at[pl.ds(ch * (r_chunk * _K), r_chunk * _K)]],
                rows_v.at[buf], sem)

        def wait_gather(buf, sem):
            pltpu.make_async_copy(g_hbm.at[pl.ds(0, r_chunk * _K)],
                                  rows_v.at[buf], sem).wait()

        def compute_store(ch, buf, sem):
            nch = o // nlane
            for r in range(r_chunk):
                accs = [plsc.bitcast(
                            rows_v[buf, r * _K, pl.ds(cc * nlane, nlane)],
                            jnp.bfloat16)
                        for cc in range(nch)]
                for t in range(1, _K):
                    for cc in range(nch):
                        accs[cc] = jnp.maximum(
                            accs[cc],
                            plsc.bitcast(
                                rows_v[buf, r * _K + t,
                                       pl.ds(cc * nlane, nlane)],
                                jnp.bfloat16))
                for cc in range(nch):
                    out_v[buf, r, pl.ds(cc * nlane, nlane)] = plsc.bitcast(
                        accs[cc], jnp.int32)
            pltpu.async_copy(out_v.at[buf],
                             m_hbm.at[pl.ds(base + ch * r_chunk, r_chunk)],
                             sem)

        def wait_store(buf, sem):
            pltpu.make_async_copy(out_v.at[buf],
                                  m_hbm.at[pl.ds(base, r_chunk)], sem).wait()

        n_half = n_chunks // 2
        start_gather(0, 0, gsem0)

        @pl.loop(0, n_half)
        def _(p):
            ch0 = p * 2

            start_gather(ch0 + 1, 1, gsem1)
            wait_gather(0, gsem0)

            @pl.when(p > 0)
            def _():
                wait_store(0, ssem0)

            compute_store(ch0, 0, ssem0)

            @pl.when(p < n_half - 1)
            def _():
                start_gather(ch0 + 2, 0, gsem0)

            wait_gather(1, gsem1)

            @pl.when(p > 0)
            def _():
                wait_store(1, ssem1)

            compute_store(ch0 + 1, 1, ssem1)

        wait_store(0, ssem0)
        wait_store(1, ssem1)

    return gather_max


def kernel(dense_fea, W, b):
    bs, emb, n_stk, n_stk_pnt = dense_fea.shape
    n = n_stk * n_stk_pnt
    o = W.shape[0]
    x = dense_fea.reshape(bs, emb, n)
    b2 = b.reshape(o, 1)
    splits = 4
    bsz = bs // splits
    parts = []
    for s in range(splits):
        xs = x[s * bsz:(s + 1) * bsz]
        idx, gw, c = _prep_call(xs, W, b2)
        idx_flat = idx[:, :, :_K].reshape(bsz * n * _K)
        m_pack = _gather_max_call(bsz * n, o // 2)(
            gw.reshape(bsz * n, o // 2), idx_flat)
        parts.append(_epi_call(m_pack.reshape(bsz, n, o // 2), c))
    out = jnp.concatenate(parts, axis=0)
    return out.reshape(bs, o, n_stk, n_stk_pnt)
